# baseline (speedup 1.0000x reference)
import jax
import jax.numpy as jnp
from jax import lax
from jax.experimental import pallas as pl
from jax.experimental.pallas import tpu as pltpu
from jax.experimental.pallas import tpu_sc as plsc

N = 10000
E = 160000
D = 256
DH = 128
NPAD = 10240
NC = 2
NS = 16
SEG = NPAD // NS

K1K = 40
K1CH = (E // (NC * NS)) // K1K

K3K = 80
K3CH = (E // NS) // K3K

BN_EPS_ = 1e-5
RBLK = 400
NBLK = N // RBLK


def _sc_mesh():
    return plsc.VectorSubcoreMesh(
        core_axis_name="c", subcore_axis_name="s", num_cores=NC, num_subcores=NS
    )


def _k1_body(dst_hbm, out_hbm, ones_v, zb_v, dstc_v,
             sm0, sm1, sm2, sm3, deg_sh):
    c = lax.axis_index("c")
    s = lax.axis_index("s")
    w = c * NS + s
    epp = E // (NC * NS)
    sems = (sm0, sm1, sm2, sm3)

    def fill(i, carry):
        zb_v[i] = jnp.zeros((16,), jnp.float32)
        ones_v[i] = jnp.ones((16,), jnp.float32)
        return carry

    lax.fori_loop(0, K1K, fill, 0)

    for r in range(SEG // K1K):
        pltpu.sync_copy(zb_v, deg_sh.at[pl.ds(s * SEG + r * K1K, K1K)])

    pltpu.sync_copy(dst_hbm.at[pl.ds(w * epp, epp)], dstc_v)
    plsc.subcore_barrier()

    def start_s(j, b):
        pltpu.async_copy(
            ones_v, deg_sh.at[dstc_v.at[pl.ds(j * K1K, K1K)]], sems[b],
            add=True,
        )

    def wait_s(j, b):
        pltpu.make_async_copy(
            ones_v, deg_sh.at[dstc_v.at[pl.ds(j * K1K, K1K)]], sems[b]
        ).wait()

    def body(g, carry):
        for b in range(4):
            j = g * 4 + b

            @pl.when(j >= 4)
            def _():
                wait_s(j - 4, b)

            start_s(j, b)
        return carry

    lax.fori_loop(0, K1CH // 4, body, 0)
    jl = K1CH - 1
    wait_s(jl - 4, 0)
    start_s(jl, 0)
    for j in range(K1CH - 4, K1CH):
        wait_s(j, j % 4)
    plsc.subcore_barrier()
    pltpu.sync_copy(
        deg_sh.at[pl.ds(s * SEG, SEG)], out_hbm.at[c, pl.ds(s * SEG, SEG)]
    )


def _k1_call(dst):
    kfn = pl.kernel(
        _k1_body,
        out_type=jax.ShapeDtypeStruct((NC, NPAD, 16), jnp.float32),
        mesh=_sc_mesh(),
        compiler_params=pltpu.CompilerParams(use_tc_tiling_on_sc=False),
        scratch_types=[
            pltpu.VMEM((K1K, 16), jnp.float32),
            pltpu.VMEM((K1K, 16), jnp.float32),
            pltpu.VMEM((E // (NC * NS),), jnp.int32),
            pltpu.SemaphoreType.DMA,
            pltpu.SemaphoreType.DMA,
            pltpu.SemaphoreType.DMA,
            pltpu.SemaphoreType.DMA,
            pltpu.VMEM_SHARED((NPAD, 16), jnp.float32),
        ],
    )
    return kfn(dst)


def _k2_body(x_ref, w_ref, p_ref, hp_ref):
    h = jnp.dot(x_ref[...], w_ref[...], preferred_element_type=jnp.float32)
    p = p_ref[...]
    deg = p[0] + p[1] + 1.0
    dinv = lax.rsqrt(deg[:, 0:1])
    hp_ref[0] = (h[:, :DH] * dinv).astype(jnp.bfloat16)
    hp_ref[1] = (h[:, DH:] * dinv).astype(jnp.bfloat16)


def _k2_call(x, W, partials):
    return pl.pallas_call(
        _k2_body,
        grid=(NBLK,),
        in_specs=[
            pl.BlockSpec((RBLK, D), lambda i: (i, 0)),
            pl.BlockSpec((D, D), lambda i: (0, 0)),
            pl.BlockSpec((NC, RBLK, 16), lambda i: (0, i, 0)),
        ],
        out_specs=pl.BlockSpec((NC, RBLK, DH), lambda i: (0, i, 0)),
        out_shape=jax.ShapeDtypeStruct((NC, N, DH), jnp.bfloat16),
    )(x, W, partials)


def _k3_body(hp_hbm, src_hbm, dst_hbm, out_hbm,
             srcl_v, dstl_v, rows0, rows1, rows2, rows3, rows4,
             gsem0, gsem1, gsem2, gsem3, gsem4,
             ssem0, ssem1, ssem2, ssem3, ssem4,
             acc_sh):
    c = lax.axis_index("c")
    s = lax.axis_index("s")
    epp = E // NS

    rows = (rows0, rows1, rows2, rows3, rows4)
    gsems = (gsem0, gsem1, gsem2, gsem3, gsem4)
    ssems = (ssem0, ssem1, ssem2, ssem3, ssem4)

    def zf(i, carry):
        for q in range(DH // 32):
            rows0[i, pl.ds(q * 32, 32)] = jnp.zeros((32,), jnp.bfloat16)
        return carry

    lax.fori_loop(0, K3K, zf, 0)
    for r in range(SEG // K3K):
        pltpu.sync_copy(rows0, acc_sh.at[pl.ds(s * SEG + r * K3K, K3K)])

    pltpu.sync_copy(src_hbm.at[pl.ds(s * epp, epp)], srcl_v)
    pltpu.sync_copy(dst_hbm.at[pl.ds(s * epp, epp)], dstl_v)
    off = c * N

    def adj(j, carry):
        srcl_v[pl.ds(j * 16, 16)] = srcl_v[pl.ds(j * 16, 16)] + off
        return carry

    lax.fori_loop(0, epp // 16, adj, 0)
    plsc.subcore_barrier()

    def start_g(j, b):
        pltpu.async_copy(
            hp_hbm.at[srcl_v.at[pl.ds(j * K3K, K3K)]], rows[b], gsems[b]
        )

    def wait_g(j, b):
        pltpu.make_async_copy(
            hp_hbm.at[srcl_v.at[pl.ds(j * K3K, K3K)]], rows[b], gsems[b]
        ).wait()

    def start_s(j, b):
        pltpu.async_copy(
            rows[b], acc_sh.at[dstl_v.at[pl.ds(j * K3K, K3K)]], ssems[b],
            add=True,
        )

    def wait_s(j, b):
        pltpu.make_async_copy(
            rows[b], acc_sh.at[dstl_v.at[pl.ds(j * K3K, K3K)]], ssems[b]
        ).wait()

    start_g(0, 0)
    start_g(1, 1)

    def gbody(g, carry):
        for b in range(5):
            j = g * 5 + b
            wait_g(j, b)
            start_s(j, b)
            bn = (b + 2) % 5

            @pl.when(j >= 3)
            def _():
                wait_s(j - 3, bn)

            @pl.when(j + 2 < K3CH)
            def _():
                start_g(j + 2, bn)
        return carry

    lax.fori_loop(0, K3CH // 5, gbody, 0)
    wait_s(K3CH - 3, (K3CH - 3) % 5)
    wait_s(K3CH - 2, (K3CH - 2) % 5)
    wait_s(K3CH - 1, (K3CH - 1) % 5)

    plsc.subcore_barrier()
    pltpu.sync_copy(
        acc_sh.at[pl.ds(s * SEG, SEG)], out_hbm.at[c, pl.ds(s * SEG, SEG)]
    )


def _k3_call(hp2, src, dst3):
    kfn = pl.kernel(
        _k3_body,
        out_type=jax.ShapeDtypeStruct((NC, NPAD, DH), jnp.bfloat16),
        mesh=_sc_mesh(),
        compiler_params=pltpu.CompilerParams(use_tc_tiling_on_sc=False),
        scratch_types=[
            pltpu.VMEM((E // NS,), jnp.int32),
            pltpu.VMEM((E // NS,), jnp.int32),
            pltpu.VMEM((K3K, DH), jnp.bfloat16),
            pltpu.VMEM((K3K, DH), jnp.bfloat16),
            pltpu.VMEM((K3K, DH), jnp.bfloat16),
            pltpu.VMEM((K3K, DH), jnp.bfloat16),
            pltpu.VMEM((K3K, DH), jnp.bfloat16),
            pltpu.SemaphoreType.DMA,
            pltpu.SemaphoreType.DMA,
            pltpu.SemaphoreType.DMA,
            pltpu.SemaphoreType.DMA,
            pltpu.SemaphoreType.DMA,
            pltpu.SemaphoreType.DMA,
            pltpu.SemaphoreType.DMA,
            pltpu.SemaphoreType.DMA,
            pltpu.SemaphoreType.DMA,
            pltpu.SemaphoreType.DMA,
            pltpu.VMEM_SHARED((NPAD, DH), jnp.bfloat16),
        ],
    )
    return kfn(hp2, src, dst3)


def _dinv_of(p):
    deg = p[0] + p[1] + 1.0
    return lax.rsqrt(deg[:, 0:1])


def _t_block(acc_ref, hp_ref, p_ref):
    dinv = _dinv_of(p_ref[...])
    a = acc_ref[...].astype(jnp.float32)
    hp = hp_ref[...].astype(jnp.float32)
    return jnp.concatenate([a[0] + hp[0], a[1] + hp[1]], axis=1) * dinv


def _d1_body(acc_ref, hp_ref, p_ref, t_ref, stats_ref):
    i = pl.program_id(0)
    tb = _t_block(acc_ref, hp_ref, p_ref)
    t_ref[...] = tb.astype(jnp.bfloat16)

    @pl.when(i == 0)
    def _():
        stats_ref[...] = jnp.zeros_like(stats_ref)

    stats_ref[...] += jnp.stack([jnp.sum(tb, 0), jnp.sum(tb * tb, 0)], 0)


def _d1_call(acc, hp, partials):
    return pl.pallas_call(
        _d1_body,
        grid=(NBLK,),
        in_specs=[
            pl.BlockSpec((NC, RBLK, DH), lambda i: (0, i, 0)),
            pl.BlockSpec((NC, RBLK, DH), lambda i: (0, i, 0)),
            pl.BlockSpec((NC, RBLK, 16), lambda i: (0, i, 0)),
        ],
        out_specs=[
            pl.BlockSpec((RBLK, D), lambda i: (i, 0)),
            pl.BlockSpec((2, D), lambda i: (0, 0)),
        ],
        out_shape=[
            jax.ShapeDtypeStruct((N, D), jnp.bfloat16),
            jax.ShapeDtypeStruct((2, D), jnp.float32),
        ],
    )(acc, hp, partials)


def _d2_body(t_ref, stats_ref, g_ref, b_ref, y_ref):
    tb = t_ref[...].astype(jnp.float32)
    st = stats_ref[...]
    mean = st[0:1] * (1.0 / N)
    ex2 = st[1:2] * (1.0 / N)
    var = ex2 - mean * mean
    scale = lax.rsqrt(var + BN_EPS_) * g_ref[...]
    y = (tb - mean) * scale + b_ref[...]
    y_ref[...] = jnp.maximum(y, 0.0)


def _d2_call(t, stats, gamma, beta):
    return pl.pallas_call(
        _d2_body,
        grid=(NBLK,),
        in_specs=[
            pl.BlockSpec((RBLK, D), lambda i: (i, 0)),
            pl.BlockSpec((2, D), lambda i: (0, 0)),
            pl.BlockSpec((1, D), lambda i: (0, 0)),
            pl.BlockSpec((1, D), lambda i: (0, 0)),
        ],
        out_specs=pl.BlockSpec((RBLK, D), lambda i: (i, 0)),
        out_shape=jax.ShapeDtypeStruct((N, D), jnp.float32),
    )(t, stats, gamma, beta)


def kernel(x, edge_index, W, bn_gamma, bn_beta):
    ei = edge_index.astype(jnp.int32)
    src = ei[0]
    dst = ei[1]

    partials = _k1_call(dst)
    hp = _k2_call(x, W, partials)
    acc = _k3_call(hp.reshape(NC * N, DH), src, dst)
    t, stats = _d1_call(acc, hp, partials)
    return _d2_call(t, stats, bn_gamma.reshape(1, D), bn_beta.reshape(1, D))


# speedup vs baseline: 1.0739x; 1.0739x over previous
import jax
import jax.numpy as jnp
from jax import lax
from jax.experimental import pallas as pl
from jax.experimental.pallas import tpu as pltpu
from jax.experimental.pallas import tpu_sc as plsc

N = 10000
E = 160000
D = 256
DH = 128
NPAD = 10240
NC = 2
NS = 16
SEG = NPAD // NS

K1K = 40
K1CH = (E // (NC * NS)) // K1K

K3K = 80
K3CH = (E // NS) // K3K

BN_EPS_ = 1e-5
RBLK = 400
NBLK = N // RBLK


def _sc_mesh():
    return plsc.VectorSubcoreMesh(
        core_axis_name="c", subcore_axis_name="s", num_cores=NC, num_subcores=NS
    )


def _k1_body(dst_hbm, out_hbm, ones_v, zb_v, dstc_v,
             sm0, sm1, sm2, sm3, deg_sh):
    c = lax.axis_index("c")
    s = lax.axis_index("s")
    w = c * NS + s
    epp = E // (NC * NS)
    sems = (sm0, sm1, sm2, sm3)

    def fill(i, carry):
        zb_v[i] = jnp.zeros((16,), jnp.float32)
        ones_v[i] = jnp.ones((16,), jnp.float32)
        return carry

    lax.fori_loop(0, K1K, fill, 0)

    for r in range(SEG // K1K):
        pltpu.sync_copy(zb_v, deg_sh.at[pl.ds(s * SEG + r * K1K, K1K)])

    pltpu.sync_copy(dst_hbm.at[pl.ds(w * epp, epp)], dstc_v)
    plsc.subcore_barrier()

    def start_s(j, b):
        pltpu.async_copy(
            ones_v, deg_sh.at[dstc_v.at[pl.ds(j * K1K, K1K)]], sems[b],
            add=True,
        )

    def wait_s(j, b):
        pltpu.make_async_copy(
            ones_v, deg_sh.at[dstc_v.at[pl.ds(j * K1K, K1K)]], sems[b]
        ).wait()

    def body(g, carry):
        for b in range(4):
            j = g * 4 + b

            @pl.when(j >= 4)
            def _():
                wait_s(j - 4, b)

            start_s(j, b)
        return carry

    lax.fori_loop(0, K1CH // 4, body, 0)
    jl = K1CH - 1
    wait_s(jl - 4, 0)
    start_s(jl, 0)
    for j in range(K1CH - 4, K1CH):
        wait_s(j, j % 4)
    plsc.subcore_barrier()
    pltpu.sync_copy(
        deg_sh.at[pl.ds(s * SEG, SEG)], out_hbm.at[c, pl.ds(s * SEG, SEG)]
    )


def _k1_call(dst):
    kfn = pl.kernel(
        _k1_body,
        out_type=jax.ShapeDtypeStruct((NC, NPAD, 16), jnp.float32),
        mesh=_sc_mesh(),
        compiler_params=pltpu.CompilerParams(use_tc_tiling_on_sc=False),
        scratch_types=[
            pltpu.VMEM((K1K, 16), jnp.float32),
            pltpu.VMEM((K1K, 16), jnp.float32),
            pltpu.VMEM((E // (NC * NS),), jnp.int32),
            pltpu.SemaphoreType.DMA,
            pltpu.SemaphoreType.DMA,
            pltpu.SemaphoreType.DMA,
            pltpu.SemaphoreType.DMA,
            pltpu.VMEM_SHARED((NPAD, 16), jnp.float32),
        ],
    )
    return kfn(dst)


def _k2_body(x_ref, w_ref, p_ref, hp_ref):
    h = jnp.dot(x_ref[...], w_ref[...], preferred_element_type=jnp.float32)
    p = p_ref[...]
    deg = p[0] + p[1] + 1.0
    dinv = lax.rsqrt(deg[:, 0:1])
    hp_ref[0] = (h[:, :DH] * dinv).astype(jnp.bfloat16)
    hp_ref[1] = (h[:, DH:] * dinv).astype(jnp.bfloat16)


def _k2_call(x, W, partials):
    return pl.pallas_call(
        _k2_body,
        grid=(NBLK,),
        in_specs=[
            pl.BlockSpec((RBLK, D), lambda i: (i, 0)),
            pl.BlockSpec((D, D), lambda i: (0, 0)),
            pl.BlockSpec((NC, RBLK, 16), lambda i: (0, i, 0)),
        ],
        out_specs=pl.BlockSpec((NC, RBLK, DH), lambda i: (0, i, 0)),
        out_shape=jax.ShapeDtypeStruct((NC, N, DH), jnp.bfloat16),
    )(x, W, partials)


def _k3_body(hp_hbm, src_hbm, dst_hbm, out_hbm,
             srcl_v, dstl_v, rows0, rows1, rows2, rows3, rows4,
             gsem0, gsem1, gsem2, gsem3, gsem4,
             ssem0, ssem1, ssem2, ssem3, ssem4,
             acc_sh):
    c = lax.axis_index("c")
    s = lax.axis_index("s")
    epp = E // NS

    rows = (rows0, rows1, rows2, rows3, rows4)
    gsems = (gsem0, gsem1, gsem2, gsem3, gsem4)
    ssems = (ssem0, ssem1, ssem2, ssem3, ssem4)

    def zf(i, carry):
        for q in range(DH // 32):
            rows0[i, pl.ds(q * 32, 32)] = jnp.zeros((32,), jnp.bfloat16)
        return carry

    lax.fori_loop(0, K3K, zf, 0)
    for r in range(SEG // K3K):
        pltpu.sync_copy(rows0, acc_sh.at[pl.ds(s * SEG + r * K3K, K3K)])

    pltpu.sync_copy(src_hbm.at[pl.ds(s * epp, epp)], srcl_v)
    pltpu.sync_copy(dst_hbm.at[pl.ds(s * epp, epp)], dstl_v)
    off = c * N

    def adj(j, carry):
        srcl_v[pl.ds(j * 16, 16)] = srcl_v[pl.ds(j * 16, 16)] + off
        return carry

    lax.fori_loop(0, epp // 16, adj, 0)
    plsc.subcore_barrier()

    def start_g(j, b):
        pltpu.async_copy(
            hp_hbm.at[srcl_v.at[pl.ds(j * K3K, K3K)]], rows[b], gsems[b]
        )

    def wait_g(j, b):
        pltpu.make_async_copy(
            hp_hbm.at[srcl_v.at[pl.ds(j * K3K, K3K)]], rows[b], gsems[b]
        ).wait()

    def start_s(j, b):
        pltpu.async_copy(
            rows[b], acc_sh.at[dstl_v.at[pl.ds(j * K3K, K3K)]], ssems[b],
            add=True,
        )

    def wait_s(j, b):
        pltpu.make_async_copy(
            rows[b], acc_sh.at[dstl_v.at[pl.ds(j * K3K, K3K)]], ssems[b]
        ).wait()

    start_g(0, 0)
    start_g(1, 1)
    start_g(2, 2)

    def gbody(g, carry):
        for b in range(5):
            j = g * 5 + b
            wait_g(j, b)
            start_s(j, b)
            bn = (b + 3) % 5

            @pl.when(j >= 2)
            def _():
                wait_s(j - 2, bn)

            @pl.when(j + 3 < K3CH)
            def _():
                start_g(j + 3, bn)
        return carry

    lax.fori_loop(0, K3CH // 5, gbody, 0)
    wait_s(K3CH - 2, (K3CH - 2) % 5)
    wait_s(K3CH - 1, (K3CH - 1) % 5)

    plsc.subcore_barrier()
    pltpu.sync_copy(
        acc_sh.at[pl.ds(s * SEG, SEG)], out_hbm.at[c, pl.ds(s * SEG, SEG)]
    )


def _k3_call(hp2, src, dst3):
    kfn = pl.kernel(
        _k3_body,
        out_type=jax.ShapeDtypeStruct((NC, NPAD, DH), jnp.bfloat16),
        mesh=_sc_mesh(),
        compiler_params=pltpu.CompilerParams(use_tc_tiling_on_sc=False),
        scratch_types=[
            pltpu.VMEM((E // NS,), jnp.int32),
            pltpu.VMEM((E // NS,), jnp.int32),
            pltpu.VMEM((K3K, DH), jnp.bfloat16),
            pltpu.VMEM((K3K, DH), jnp.bfloat16),
            pltpu.VMEM((K3K, DH), jnp.bfloat16),
            pltpu.VMEM((K3K, DH), jnp.bfloat16),
            pltpu.VMEM((K3K, DH), jnp.bfloat16),
            pltpu.SemaphoreType.DMA,
            pltpu.SemaphoreType.DMA,
            pltpu.SemaphoreType.DMA,
            pltpu.SemaphoreType.DMA,
            pltpu.SemaphoreType.DMA,
            pltpu.SemaphoreType.DMA,
            pltpu.SemaphoreType.DMA,
            pltpu.SemaphoreType.DMA,
            pltpu.SemaphoreType.DMA,
            pltpu.SemaphoreType.DMA,
            pltpu.VMEM_SHARED((NPAD, DH), jnp.bfloat16),
        ],
    )
    return kfn(hp2, src, dst3)


def _dinv_of(p):
    deg = p[0] + p[1] + 1.0
    return lax.rsqrt(deg[:, 0:1])


def _t_block(acc_ref, hp_ref, p_ref):
    dinv = _dinv_of(p_ref[...])
    a = acc_ref[...].astype(jnp.float32)
    hp = hp_ref[...].astype(jnp.float32)
    return jnp.concatenate([a[0] + hp[0], a[1] + hp[1]], axis=1) * dinv


def _d1_body(acc_ref, hp_ref, p_ref, t_ref, stats_ref):
    i = pl.program_id(0)
    tb = _t_block(acc_ref, hp_ref, p_ref)
    t_ref[...] = tb.astype(jnp.bfloat16)

    @pl.when(i == 0)
    def _():
        stats_ref[...] = jnp.zeros_like(stats_ref)

    stats_ref[...] += jnp.stack([jnp.sum(tb, 0), jnp.sum(tb * tb, 0)], 0)


def _d1_call(acc, hp, partials):
    return pl.pallas_call(
        _d1_body,
        grid=(NBLK,),
        in_specs=[
            pl.BlockSpec((NC, RBLK, DH), lambda i: (0, i, 0)),
            pl.BlockSpec((NC, RBLK, DH), lambda i: (0, i, 0)),
            pl.BlockSpec((NC, RBLK, 16), lambda i: (0, i, 0)),
        ],
        out_specs=[
            pl.BlockSpec((RBLK, D), lambda i: (i, 0)),
            pl.BlockSpec((2, D), lambda i: (0, 0)),
        ],
        out_shape=[
            jax.ShapeDtypeStruct((N, D), jnp.bfloat16),
            jax.ShapeDtypeStruct((2, D), jnp.float32),
        ],
    )(acc, hp, partials)


def _d2_body(t_ref, stats_ref, g_ref, b_ref, y_ref):
    tb = t_ref[...].astype(jnp.float32)
    st = stats_ref[...]
    mean = st[0:1] * (1.0 / N)
    ex2 = st[1:2] * (1.0 / N)
    var = ex2 - mean * mean
    scale = lax.rsqrt(var + BN_EPS_) * g_ref[...]
    y = (tb - mean) * scale + b_ref[...]
    y_ref[...] = jnp.maximum(y, 0.0)


def _d2_call(t, stats, gamma, beta):
    return pl.pallas_call(
        _d2_body,
        grid=(NBLK,),
        in_specs=[
            pl.BlockSpec((RBLK, D), lambda i: (i, 0)),
            pl.BlockSpec((2, D), lambda i: (0, 0)),
            pl.BlockSpec((1, D), lambda i: (0, 0)),
            pl.BlockSpec((1, D), lambda i: (0, 0)),
        ],
        out_specs=pl.BlockSpec((RBLK, D), lambda i: (i, 0)),
        out_shape=jax.ShapeDtypeStruct((N, D), jnp.float32),
    )(t, stats, gamma, beta)


def kernel(x, edge_index, W, bn_gamma, bn_beta):
    ei = edge_index.astype(jnp.int32)
    src = ei[0]
    dst = ei[1]

    partials = _k1_call(dst)
    hp = _k2_call(x, W, partials)
    acc = _k3_call(hp.reshape(NC * N, DH), src, dst)
    t, stats = _d1_call(acc, hp, partials)
    return _d2_call(t, stats, bn_gamma.reshape(1, D), bn_beta.reshape(1, D))
